# baseline (speedup 1.0000x reference)
import jax
import jax.numpy as jnp
from jax.experimental import pallas as pl
from jax.experimental.pallas import tpu as pltpu

_B, _S, _D = 4, 4096, 1024
_CH = 2048
_NSC = _S // _CH
_NT = _NSC * _B
_NBUF = 4


def _pipe_kernel(x_hbm, pos_hbm, o_hbm, xb, pb,
                 sx0, sx1, sx2, sx3, sp0, sp1, so0, so1, so2, so3):
    sx = (sx0, sx1, sx2, sx3)
    sp = (sp0, sp1)
    so = (so0, so1, so2, so3)

    def xrow0(t):
        sc, b = divmod(t, _B)
        return b * _S + sc * _CH

    dx = [None] * _NT
    dp = [None] * _NSC
    do = [None] * _NT

    def start_in(t):
        i = t % _NBUF
        d = pltpu.make_async_copy(
            x_hbm.at[pl.ds(xrow0(t), _CH)], xb.at[i], sx[i])
        d.start()
        dx[t] = d
        sc, b = divmod(t, _B)
        if b == 0:
            j = sc % 2
            d = pltpu.make_async_copy(
                pos_hbm.at[pl.ds(sc * _CH, _CH)], pb.at[j], sp[j])
            d.start()
            dp[sc] = d

    start_in(0)
    start_in(1)
    for t in range(_NT):
        i = t % _NBUF
        if t + 2 < _NT:
            if t - 2 >= 0:
                do[t - 2].wait()
            start_in(t + 2)
        dx[t].wait()
        sc, b = divmod(t, _B)
        if b == 0:
            dp[sc].wait()
        xv = xb.at[i]
        xv[...] = xv[...] + pb[sc % 2]
        d = pltpu.make_async_copy(xv, o_hbm.at[pl.ds(xrow0(t), _CH)], so[i])
        d.start()
        do[t] = d
    for t in range(_NT - 4, _NT):
        do[t].wait()


def kernel(x, pos_table):
    B, S, D = x.shape
    x2d = x.reshape(B * S, D)
    out2d = pl.pallas_call(
        _pipe_kernel,
        in_specs=[
            pl.BlockSpec(memory_space=pltpu.MemorySpace.HBM),
            pl.BlockSpec(memory_space=pltpu.MemorySpace.HBM),
        ],
        out_specs=pl.BlockSpec(memory_space=pltpu.MemorySpace.HBM),
        out_shape=jax.ShapeDtypeStruct((B * S, D), x.dtype),
        scratch_shapes=[
            pltpu.VMEM((_NBUF, _CH, D), jnp.float32),
            pltpu.VMEM((2, _CH, D), jnp.float32),
            pltpu.SemaphoreType.DMA,
            pltpu.SemaphoreType.DMA,
            pltpu.SemaphoreType.DMA,
            pltpu.SemaphoreType.DMA,
            pltpu.SemaphoreType.DMA,
            pltpu.SemaphoreType.DMA,
            pltpu.SemaphoreType.DMA,
            pltpu.SemaphoreType.DMA,
            pltpu.SemaphoreType.DMA,
            pltpu.SemaphoreType.DMA,
        ],
    )(x2d, pos_table)
    return out2d.reshape(B, S, D)
